# Initial kernel scaffold; baseline (speedup 1.0000x reference)
import jax
import jax.numpy as jnp
from jax.experimental import pallas as pl


def kernel(x, src0, dst0, src1, dst1, n_dst0, n_dst1, W_self1, W_neigh1, b1, W_self2, W_neigh2, b2):
    raise NotImplementedError("write your pallas kernel here")


# speedup vs baseline: 3.2120x; 3.2120x over previous
import functools

import jax
import jax.numpy as jnp
from jax import lax
from jax.experimental import pallas as pl
from jax.experimental.pallas import tpu as pltpu
from jax.experimental.pallas import tpu_sc as plsc

_NC = 2
_NS = 16
_LANES = 16
_CHUNK = 128
_DEGW = 16
_HALF = 64


@functools.lru_cache(maxsize=None)
def _sc_agg_call(n_half_rows, n_edges, n_dst):
    ept = n_edges // _NS
    assert ept % _CHUNK == 0 and n_edges == ept * _NS
    nchunks = ept // _CHUNK
    rpt = n_dst // _NS
    assert rpt * _NS == n_dst
    zc = min(128, rpt)
    assert rpt % zc == 0
    mesh = plsc.VectorSubcoreMesh(core_axis_name="c", subcore_axis_name="s")

    def body(x_hbm, src_hbm, dst_hbm, z64_hbm, z16_hbm, o16_hbm,
             agg_hbm, deg_hbm,
             sidx, didx, rows, onesv, zrow, z16v, acc, dacc, sem):
        c = lax.axis_index("c")
        s = lax.axis_index("s")
        pltpu.sync_copy(z64_hbm, zrow)
        pltpu.sync_copy(z16_hbm, z16v)
        pltpu.sync_copy(o16_hbm, onesv)

        def zbody(j, carry):
            base = s * rpt + j * zc
            pltpu.sync_copy(zrow.at[pl.ds(0, zc)], acc.at[pl.ds(base, zc)])
            pltpu.sync_copy(z16v.at[pl.ds(0, zc)], dacc.at[pl.ds(base, zc)])
            return carry
        lax.fori_loop(0, rpt // zc, zbody, 0)
        plsc.subcore_barrier()

        def ebody(i, carry):
            e0 = s * ept + i * _CHUNK
            pltpu.sync_copy(src_hbm.at[pl.ds(e0, _CHUNK)], sidx)
            pltpu.sync_copy(dst_hbm.at[pl.ds(e0, _CHUNK)], didx)
            for k in range(_CHUNK // _LANES):
                sl = pl.ds(k * _LANES, _LANES)
                sidx[sl] = sidx[sl] * 2 + c
            pltpu.async_copy(x_hbm.at[sidx], rows, sem).wait()
            pltpu.sync_copy(rows, acc.at[didx], add=True)

            @pl.when(c == 0)
            def _():
                pltpu.sync_copy(onesv, dacc.at[didx], add=True)
            return carry
        lax.fori_loop(0, nchunks, ebody, 0)
        plsc.subcore_barrier()

        base = s * rpt
        pltpu.sync_copy(acc.at[pl.ds(base, rpt)],
                        agg_hbm.at[c, pl.ds(base, rpt)])

        @pl.when(c == 0)
        def _():
            pltpu.sync_copy(dacc.at[pl.ds(base, rpt)],
                            deg_hbm.at[pl.ds(base, rpt)])

    return pl.kernel(
        body,
        out_type=[
            jax.ShapeDtypeStruct((_NC, n_dst, _HALF), jnp.float32),
            jax.ShapeDtypeStruct((n_dst, _DEGW), jnp.float32),
        ],
        mesh=mesh,
        scratch_types=[
            pltpu.VMEM((_CHUNK,), jnp.int32),
            pltpu.VMEM((_CHUNK,), jnp.int32),
            pltpu.VMEM((_CHUNK, _HALF), jnp.float32),
            pltpu.VMEM((_CHUNK, _DEGW), jnp.float32),
            pltpu.VMEM((128, _HALF), jnp.float32),
            pltpu.VMEM((128, _DEGW), jnp.float32),
            pltpu.VMEM_SHARED((n_dst, _HALF), jnp.float32),
            pltpu.VMEM_SHARED((n_dst, _DEGW), jnp.float32),
            pltpu.SemaphoreType.DMA,
        ],
        compiler_params=pltpu.CompilerParams(use_tc_tiling_on_sc=False),
    )


@functools.lru_cache(maxsize=None)
def _dense_call(n_rows, relu, blk):
    def body(xd, a_a, a_b, dg, ws, wnt, wnb, b, out):
        m = (jnp.dot(a_a[...], wnt[...], preferred_element_type=jnp.float32)
             + jnp.dot(a_b[...], wnb[...], preferred_element_type=jnp.float32))
        deg = jnp.maximum(dg[...][:, 0:1], 1.0)
        r = (jnp.dot(xd[...], ws[...], preferred_element_type=jnp.float32)
             + m / deg + b[...])
        out[...] = jnp.maximum(r, 0.0) if relu else r

    return pl.pallas_call(
        body,
        grid=(n_rows // blk,),
        in_specs=[
            pl.BlockSpec((blk, 128), lambda i: (i, 0)),
            pl.BlockSpec((blk, _HALF), lambda i: (i, 0)),
            pl.BlockSpec((blk, _HALF), lambda i: (i, 0)),
            pl.BlockSpec((blk, _DEGW), lambda i: (i, 0)),
            pl.BlockSpec((128, 128), lambda i: (0, 0)),
            pl.BlockSpec((_HALF, 128), lambda i: (0, 0)),
            pl.BlockSpec((_HALF, 128), lambda i: (0, 0)),
            pl.BlockSpec((1, 128), lambda i: (0, 0)),
        ],
        out_specs=pl.BlockSpec((blk, 128), lambda i: (i, 0)),
        out_shape=jax.ShapeDtypeStruct((n_rows, 128), jnp.float32),
    )


def kernel(x, src0, dst0, src1, dst1, n_dst0, n_dst1,
           W_self1, W_neigh1, b1, W_self2, W_neigh2, b2):
    del n_dst0, n_dst1
    n1 = src1.shape[0]
    n2 = 1024
    f32 = jnp.float32
    x64 = x.reshape(-1, _HALF)
    z64 = jnp.zeros((128, _HALF), f32)
    z16 = jnp.zeros((128, _DEGW), f32)
    o16 = jnp.ones((_CHUNK, _DEGW), f32)
    src0i = src0.astype(jnp.int32)
    dst0i = dst0.astype(jnp.int32)
    src1i = src1.astype(jnp.int32)
    dst1i = dst1.astype(jnp.int32)

    agg1, deg1 = _sc_agg_call(x64.shape[0], src0.shape[0], n1)(
        x64, src0i, dst0i, z64, z16, o16)
    h1 = _dense_call(n1, True, 2048)(
        x[:n1], agg1[0], agg1[1], deg1,
        W_self1, W_neigh1[:_HALF], W_neigh1[_HALF:], b1.reshape(1, 128))
    agg2, deg2 = _sc_agg_call(2 * n1, src1.shape[0], n2)(
        h1.reshape(-1, _HALF), src1i, dst1i, z64, z16, o16)
    out = _dense_call(n2, False, 1024)(
        h1[:n2], agg2[0], agg2[1], deg2,
        W_self2, W_neigh2[:_HALF], W_neigh2[_HALF:], b2.reshape(1, 128))
    return out


# speedup vs baseline: 6.2969x; 1.9604x over previous
import functools

import jax
import jax.numpy as jnp
from jax import lax
from jax.experimental import pallas as pl
from jax.experimental.pallas import tpu as pltpu
from jax.experimental.pallas import tpu_sc as plsc

_NC = 2
_NS = 16
_LANES = 16
_CHUNK = 128
_G = 4
_DEGW = 16
_HALF = 64


@functools.lru_cache(maxsize=None)
def _sc_agg_call(n_half_rows, n_edges, n_dst):
    nchunks = n_edges // (_NS * _CHUNK)
    assert nchunks * _NS * _CHUNK == n_edges
    g = min(_G, nchunks)
    ngroups = nchunks // g
    assert ngroups * g == nchunks and (ngroups == 1 or ngroups % 2 == 0)
    rpt = n_dst // _NS
    assert rpt * _NS == n_dst
    zc = min(128, rpt)
    assert rpt % zc == 0
    mesh = plsc.VectorSubcoreMesh(core_axis_name="c", subcore_axis_name="s")

    def body(x_hbm, src_hbm, dst_hbm, z64_hbm, z16_hbm, o16_hbm,
             agg_hbm, deg_hbm,
             sidxA, didxA, sidxB, didxB, rows, onesv, zrow, z16v,
             acc, dacc, psemA, psemB, gsem, ssem, dsem):
        c = lax.axis_index("c")
        s = lax.axis_index("s")
        row0 = s * nchunks

        def start_prefetch(grp, sbuf, dbuf, sem):
            base = row0 + grp * g
            pltpu.async_copy(src_hbm.at[pl.ds(base, g)], sbuf, sem)
            pltpu.async_copy(dst_hbm.at[pl.ds(base, g)], dbuf, sem)

        def wait_prefetch(sbuf, dbuf, sem):
            pltpu.make_async_copy(src_hbm.at[pl.ds(0, g)], sbuf, sem).wait()
            pltpu.make_async_copy(dst_hbm.at[pl.ds(0, g)], dbuf, sem).wait()

        def transform(sbuf):
            for j in range(g):
                for k in range(_CHUNK // _LANES):
                    sl = pl.ds(k * _LANES, _LANES)
                    sbuf[j, sl] = sbuf[j, sl] * 2 + c

        def fire_gathers(sbuf):
            for j in range(g):
                pltpu.async_copy(x_hbm.at[sbuf.at[j]], rows.at[j], gsem)

        def drain_and_scatter(sbuf, dbuf):
            for j in range(g):
                pltpu.make_async_copy(
                    x_hbm.at[sbuf.at[j]], rows.at[j], gsem).wait()
                pltpu.async_copy(rows.at[j], acc.at[dbuf.at[j]], ssem,
                                 add=True)
                if j % 2 == 0:
                    @pl.when(c == 0)
                    def _():
                        pltpu.async_copy(onesv, dacc.at[dbuf.at[j]], dsem,
                                         add=True)
                else:
                    @pl.when(c == 1)
                    def _():
                        pltpu.async_copy(onesv, dacc.at[dbuf.at[j]], dsem,
                                         add=True)

        def drain_scatters(dbuf):
            for j in range(g):
                pltpu.make_async_copy(
                    rows.at[j], acc.at[dbuf.at[j]], ssem).wait()
            for j in range(g // 2):
                pltpu.make_async_copy(onesv, dacc.at[dbuf.at[0]], dsem).wait()

        start_prefetch(0, sidxA, didxA, psemA)
        if ngroups > 1:
            start_prefetch(1, sidxB, didxB, psemB)

        pltpu.sync_copy(z64_hbm, zrow)
        pltpu.sync_copy(z16_hbm, z16v)
        pltpu.sync_copy(o16_hbm, onesv)

        def zbody(jz, carry):
            base = s * rpt + jz * zc
            pltpu.sync_copy(zrow.at[pl.ds(0, zc)], acc.at[pl.ds(base, zc)])
            pltpu.sync_copy(z16v.at[pl.ds(0, zc)], dacc.at[pl.ds(base, zc)])
            return carry
        lax.fori_loop(0, rpt // zc, zbody, 0)
        plsc.subcore_barrier()

        wait_prefetch(sidxA, didxA, psemA)
        transform(sidxA)

        if ngroups == 1:
            fire_gathers(sidxA)
            drain_and_scatter(sidxA, didxA)
            drain_scatters(didxA)
        else:
            def half(grp, sbuf, dbuf, psem, osbuf, odbuf, opsem):
                fire_gathers(sbuf)

                @pl.when(grp + 1 < ngroups)
                def _():
                    wait_prefetch(osbuf, odbuf, opsem)
                transform(osbuf)
                drain_and_scatter(sbuf, dbuf)
                drain_scatters(dbuf)

                @pl.when(grp + 2 < ngroups)
                def _():
                    start_prefetch(grp + 2, sbuf, dbuf, psem)

            def pair_body(p, carry):
                g0 = 2 * p
                half(g0, sidxA, didxA, psemA, sidxB, didxB, psemB)
                half(g0 + 1, sidxB, didxB, psemB, sidxA, didxA, psemA)
                return carry
            lax.fori_loop(0, ngroups // 2, pair_body, 0)
        plsc.subcore_barrier()

        base = s * rpt
        pltpu.sync_copy(acc.at[pl.ds(base, rpt)],
                        agg_hbm.at[c, pl.ds(base, rpt)])
        pltpu.sync_copy(dacc.at[pl.ds(base, rpt)],
                        deg_hbm.at[c, pl.ds(base, rpt)])

    return pl.kernel(
        body,
        out_type=[
            jax.ShapeDtypeStruct((_NC, n_dst, _HALF), jnp.float32),
            jax.ShapeDtypeStruct((_NC, n_dst, _DEGW), jnp.float32),
        ],
        mesh=mesh,
        scratch_types=[
            pltpu.VMEM((g, _CHUNK), jnp.int32),
            pltpu.VMEM((g, _CHUNK), jnp.int32),
            pltpu.VMEM((g, _CHUNK), jnp.int32),
            pltpu.VMEM((g, _CHUNK), jnp.int32),
            pltpu.VMEM((g, _CHUNK, _HALF), jnp.float32),
            pltpu.VMEM((_CHUNK, _DEGW), jnp.float32),
            pltpu.VMEM((128, _HALF), jnp.float32),
            pltpu.VMEM((128, _DEGW), jnp.float32),
            pltpu.VMEM_SHARED((n_dst, _HALF), jnp.float32),
            pltpu.VMEM_SHARED((n_dst, _DEGW), jnp.float32),
            pltpu.SemaphoreType.DMA,
            pltpu.SemaphoreType.DMA,
            pltpu.SemaphoreType.DMA,
            pltpu.SemaphoreType.DMA,
            pltpu.SemaphoreType.DMA,
        ],
        compiler_params=pltpu.CompilerParams(use_tc_tiling_on_sc=False),
    )


@functools.lru_cache(maxsize=None)
def _dense_call(n_rows, relu, blk):
    def body(xd, a_a, a_b, dg_a, dg_b, ws, wnt, wnb, b, out):
        m = (jnp.dot(a_a[...], wnt[...], preferred_element_type=jnp.float32)
             + jnp.dot(a_b[...], wnb[...], preferred_element_type=jnp.float32))
        deg = jnp.maximum(dg_a[...][:, 0:1] + dg_b[...][:, 0:1], 1.0)
        r = (jnp.dot(xd[...], ws[...], preferred_element_type=jnp.float32)
             + m / deg + b[...])
        out[...] = jnp.maximum(r, 0.0) if relu else r

    return pl.pallas_call(
        body,
        grid=(n_rows // blk,),
        in_specs=[
            pl.BlockSpec((blk, 128), lambda i: (i, 0)),
            pl.BlockSpec((blk, _HALF), lambda i: (i, 0)),
            pl.BlockSpec((blk, _HALF), lambda i: (i, 0)),
            pl.BlockSpec((blk, _DEGW), lambda i: (i, 0)),
            pl.BlockSpec((blk, _DEGW), lambda i: (i, 0)),
            pl.BlockSpec((128, 128), lambda i: (0, 0)),
            pl.BlockSpec((_HALF, 128), lambda i: (0, 0)),
            pl.BlockSpec((_HALF, 128), lambda i: (0, 0)),
            pl.BlockSpec((1, 128), lambda i: (0, 0)),
        ],
        out_specs=pl.BlockSpec((blk, 128), lambda i: (i, 0)),
        out_shape=jax.ShapeDtypeStruct((n_rows, 128), jnp.float32),
    )


def kernel(x, src0, dst0, src1, dst1, n_dst0, n_dst1,
           W_self1, W_neigh1, b1, W_self2, W_neigh2, b2):
    del n_dst0, n_dst1
    n1 = src1.shape[0]
    n2 = 1024
    f32 = jnp.float32
    x64 = x.reshape(-1, _HALF)
    z64 = jnp.zeros((128, _HALF), f32)
    z16 = jnp.zeros((128, _DEGW), f32)
    o16 = jnp.ones((_CHUNK, _DEGW), f32)
    src0i = src0.astype(jnp.int32).reshape(-1, _CHUNK)
    dst0i = dst0.astype(jnp.int32).reshape(-1, _CHUNK)
    src1i = src1.astype(jnp.int32).reshape(-1, _CHUNK)
    dst1i = dst1.astype(jnp.int32).reshape(-1, _CHUNK)

    agg1, deg1 = _sc_agg_call(x64.shape[0], src0.shape[0], n1)(
        x64, src0i, dst0i, z64, z16, o16)
    h1 = _dense_call(n1, True, 2048)(
        x[:n1], agg1[0], agg1[1], deg1[0], deg1[1],
        W_self1, W_neigh1[:_HALF], W_neigh1[_HALF:], b1.reshape(1, 128))
    agg2, deg2 = _sc_agg_call(2 * n1, src1.shape[0], n2)(
        h1.reshape(-1, _HALF), src1i, dst1i, z64, z16, o16)
    out = _dense_call(n2, False, 1024)(
        h1[:n2], agg2[0], agg2[1], deg2[0], deg2[1],
        W_self2, W_neigh2[:_HALF], W_neigh2[_HALF:], b2.reshape(1, 128))
    return out
